# Initial kernel scaffold; baseline (speedup 1.0000x reference)
import jax
import jax.numpy as jnp
from jax.experimental import pallas as pl


def kernel(localizations, classifications, localizations_default):
    raise NotImplementedError("write your pallas kernel here")


# speedup vs baseline: 1.2144x; 1.2144x over previous
import jax
import jax.numpy as jnp
from jax.experimental import pallas as pl
from jax.experimental.pallas import tpu as pltpu

N_REAL = 5000
NP = 5120
NC = 11
C_PAD = 16
TI = 128
TJ = 512
CH = 128
NCH = TJ // CH
NI = NP // TI
NJ = NP // TJ
OVERLAP = 0.45
CLS_THRESH = 0.01
FAR = 1e6


def _prep_kernel(loc_ref, cls_ref, def_ref, boxes_ref, msc_ref):
    dc = def_ref[0, 0:1, :]
    dw = def_ref[0, 1:2, :]
    ctr = dc + loc_ref[0, 0:1, :] * 0.1 * dw
    w = dw * jnp.exp(loc_ref[0, 1:2, :] * 0.2)
    s = ctr - w / 2.0
    e = ctr + w / 2.0
    se = jnp.concatenate([s, e], axis=0)
    lane2 = jax.lax.broadcasted_iota(jnp.int32, (2, NP), 1)
    boxes_ref[0] = jnp.where(lane2 < N_REAL, se, FAR)

    x = cls_ref[0]
    mx = jnp.max(x, axis=0, keepdims=True)
    ex = jnp.exp(x - mx)
    sc = ex / jnp.sum(ex, axis=0, keepdims=True)
    fg = sc[1:NC, :]
    msc = jnp.where(fg > CLS_THRESH, fg, 0.0)
    lane = jax.lax.broadcasted_iota(jnp.int32, (NC - 1, NP), 1)
    msc = jnp.where(lane < N_REAL, msc, 0.0)
    msc_ref[0] = jnp.concatenate(
        [msc, jnp.zeros((C_PAD - (NC - 1), NP), jnp.float32)], axis=0)


def _nms_kernel(bi_ref, mi_ref, bj_ref, mj_ref, out_ref,
                sjb_ref, ejb_ref, ljb_ref, mscb_ref, m_ref, bnd_ref):
    j = pl.program_id(1)
    i = pl.program_id(2)

    @pl.when(i == 0)
    def _():
        sj = bj_ref[0, :, 0:1]
        ej = bj_ref[0, :, 1:2]
        sjb_ref[...] = jnp.broadcast_to(sj, (TJ, TI))
        ejb_ref[...] = jnp.broadcast_to(ej, (TJ, TI))
        ljb_ref[...] = jnp.broadcast_to((ej - sj) * OVERLAP, (TJ, TI))
        bnd_ref[0] = jnp.min(sj)
        bnd_ref[1] = jnp.max(jnp.where(sj < FAR, ej, -FAR))
        for c in range(NC - 1):
            mscb_ref[c] = jnp.broadcast_to(mj_ref[0, :, c:c + 1], (TJ, TI))

    si = bi_ref[0, 0:1, :]
    ei = bi_ref[0, 1:2, :]

    @pl.when(j == 0)
    def _():
        m_ref[i] = jnp.zeros((C_PAD, TI), jnp.float32)

    ismin = jnp.min(si)
    iemax = jnp.max(jnp.where(si < FAR, ei, -FAR))
    live = (bnd_ref[0] <= iemax) & (bnd_ref[1] >= ismin)

    @pl.when(live)
    def _():
        li45 = (ei - si) * OVERLAP

        def chunk_body(k, parts):
            r0 = k * CH
            interk = jnp.maximum(
                jnp.minimum(ejb_ref[pl.ds(r0, CH), :], ei)
                - jnp.maximum(sjb_ref[pl.ds(r0, CH), :], si), 0.0)
            afk = jnp.where(
                interk * (1.0 + OVERLAP) > (ljb_ref[pl.ds(r0, CH), :] + li45),
                1.0, 0.0)
            out = []
            for c in range(NC - 1):
                tk = afk * mscb_ref[c, pl.ds(r0, CH), :]
                pm = jnp.max(tk.reshape(CH // 8, 8, TI), axis=0)
                out.append(jnp.maximum(parts[c], pm))
            return tuple(out)

        parts0 = tuple(jnp.zeros((8, TI), jnp.float32) for _ in range(NC - 1))
        parts = jax.lax.fori_loop(0, NCH, chunk_body, parts0)
        terms = [jnp.max(p, axis=0, keepdims=True) for p in parts]
        new = jnp.concatenate(
            terms + [jnp.zeros((C_PAD - (NC - 1), TI), jnp.float32)], axis=0)
        m_ref[i] = jnp.maximum(m_ref[i], new)

    @pl.when(j == NJ - 1)
    def _():
        m = m_ref[i]
        msci = mi_ref[0]
        contrib = jnp.where(m <= msci, msci, 0.0)
        best = jnp.max(contrib, axis=0, keepdims=True)
        inr = (si > -10.0) & (ei < 10.0)
        best = jnp.where(inr, best, 0.0)
        out_ref[0] = jnp.concatenate(
            [best, jnp.zeros((7, TI), jnp.float32)], axis=0)


def kernel(localizations, classifications, localizations_default):
    B = localizations.shape[0]
    pad = NP - N_REAL

    dc = localizations_default[None, :, 0]
    dw = localizations_default[None, :, 1]
    key = (dc + localizations[:, :, 0] * 0.1 * dw
           - 0.5 * dw * jnp.exp(localizations[:, :, 1] * 0.2))
    key_p = jnp.pad(key, ((0, 0), (0, pad)), constant_values=jnp.inf)
    perm = jnp.argsort(key_p, axis=1)
    inv = jnp.argsort(perm, axis=1)

    loc_p = jnp.pad(localizations, ((0, 0), (0, pad), (0, 0)))
    cls_p = jnp.pad(classifications, ((0, 0), (0, pad), (0, 0)))
    def_p = jnp.broadcast_to(
        jnp.pad(localizations_default, ((0, pad), (0, 0)))[None], (B, NP, 2))
    loc_s = jnp.take_along_axis(loc_p, perm[:, :, None], axis=1)
    cls_s = jnp.take_along_axis(cls_p, perm[:, :, None], axis=1)
    def_s = jnp.take_along_axis(def_p, perm[:, :, None], axis=1)

    locT = loc_s.transpose(0, 2, 1)
    clsT = cls_s.transpose(0, 2, 1)
    defT = def_s.transpose(0, 2, 1)

    boxes, mscT = pl.pallas_call(
        _prep_kernel,
        grid=(B,),
        in_specs=[
            pl.BlockSpec((1, 2, NP), lambda b: (b, 0, 0)),
            pl.BlockSpec((1, NC, NP), lambda b: (b, 0, 0)),
            pl.BlockSpec((1, 2, NP), lambda b: (b, 0, 0)),
        ],
        out_specs=[
            pl.BlockSpec((1, 2, NP), lambda b: (b, 0, 0)),
            pl.BlockSpec((1, C_PAD, NP), lambda b: (b, 0, 0)),
        ],
        out_shape=[
            jax.ShapeDtypeStruct((B, 2, NP), jnp.float32),
            jax.ShapeDtypeStruct((B, C_PAD, NP), jnp.float32),
        ],
    )(locT, clsT, defT)

    boxes_j = boxes.transpose(0, 2, 1)
    msc_j = mscT.transpose(0, 2, 1)

    best8 = pl.pallas_call(
        _nms_kernel,
        grid=(B, NJ, NI),
        in_specs=[
            pl.BlockSpec((1, 2, TI), lambda b, j, i: (b, 0, i)),
            pl.BlockSpec((1, C_PAD, TI), lambda b, j, i: (b, 0, i)),
            pl.BlockSpec((1, TJ, 2), lambda b, j, i: (b, j, 0)),
            pl.BlockSpec((1, TJ, C_PAD), lambda b, j, i: (b, j, 0)),
        ],
        out_specs=pl.BlockSpec((1, 8, TI), lambda b, j, i: (b, 0, i)),
        out_shape=jax.ShapeDtypeStruct((B, 8, NP), jnp.float32),
        scratch_shapes=[
            pltpu.VMEM((TJ, TI), jnp.float32),
            pltpu.VMEM((TJ, TI), jnp.float32),
            pltpu.VMEM((TJ, TI), jnp.float32),
            pltpu.VMEM((NC - 1, TJ, TI), jnp.float32),
            pltpu.VMEM((NI, C_PAD, TI), jnp.float32),
            pltpu.SMEM((2,), jnp.float32),
        ],
    )(boxes, mscT, boxes_j, msc_j)

    dec = boxes.transpose(0, 2, 1)
    best = best8[:, 0, :]
    out_s = jnp.concatenate([dec, best[:, :, None]], axis=2)
    return jnp.take_along_axis(out_s, inv[:, :N_REAL, None], axis=1)


# speedup vs baseline: 1.2487x; 1.0283x over previous
import jax
import jax.numpy as jnp
from jax.experimental import pallas as pl
from jax.experimental.pallas import tpu as pltpu

N_REAL = 5000
NP = 5120
NC = 11
C_PAD = 16
TI = 128
TJ = 512
CH = 128
NCH = TJ // CH
NI = NP // TI
NJ = NP // TJ
OVERLAP = 0.45
CLS_THRESH = 0.01
FAR = 1e6
NB = 128


def _pos_kernel(key_ref, pos_ref, perm_ref):
    kcol = key_ref[0]
    ridx = jax.lax.broadcasted_iota(jnp.int32, (NP, 1), 0)
    valid = ridx < N_REAL
    kmin = jnp.min(jnp.where(valid, kcol, jnp.inf))
    kmax = jnp.max(jnp.where(valid, kcol, -jnp.inf))
    scale = NB / jnp.maximum(kmax - kmin, 1e-30)
    bfl = jnp.clip(jnp.floor((kcol - kmin) * scale), 0.0, NB - 1.0)
    bi = jnp.where(valid, bfl.astype(jnp.int32), NB - 1)
    lane = jax.lax.broadcasted_iota(jnp.int32, (NP, NB), 1)
    H = jnp.where(lane == bi, 1.0, 0.0)
    C = H
    sh = 1
    while sh < NP:
        C = C + jnp.concatenate(
            [jnp.zeros((sh, NB), jnp.float32), C[:NP - sh, :]], axis=0)
        sh *= 2
    tot = C[NP - 1:NP, :]
    shl = 1
    acc = tot
    while shl < NB:
        acc_shift = jnp.concatenate(
            [jnp.zeros((1, shl), jnp.float32), acc[:, :NB - shl]], axis=1)
        acc = acc + acc_shift
        shl *= 2
    off = acc - tot
    posf = jnp.sum(H * (C + off - 1.0), axis=1, keepdims=True)
    pos_ref[0] = posf.astype(jnp.int32)

    lanef = lane.astype(jnp.float32)
    kf = ridx.astype(jnp.float32)
    gk = jnp.floor(posf * (1.0 / NB))
    lk = posf - gk * NB
    W = jnp.where(lanef == gk, 1.0, 0.0)
    onel = jnp.where(lanef == lk, 1.0, 0.0)
    khi = jnp.floor(kf * (1.0 / 64.0))
    klo = kf - khi * 64.0
    dims = (((0,), (0,)), ((), ()))
    Rhi = jax.lax.dot_general(W, onel * khi, dims,
                              preferred_element_type=jnp.float32)
    Rlo = jax.lax.dot_general(W, onel * klo, dims,
                              preferred_element_type=jnp.float32)
    R = Rhi * 64.0 + Rlo
    permf = R[:NP // NB, :].reshape(1, NP)
    perm_ref[0] = permf.astype(jnp.int32)


def _prep_kernel(loc_ref, cls_ref, def_ref, boxes_ref, msc_ref):
    dc = def_ref[0, 0:1, :]
    dw = def_ref[0, 1:2, :]
    ctr = dc + loc_ref[0, 0:1, :] * 0.1 * dw
    w = dw * jnp.exp(loc_ref[0, 1:2, :] * 0.2)
    s = ctr - w / 2.0
    e = ctr + w / 2.0
    se = jnp.concatenate([s, e], axis=0)
    lane2 = jax.lax.broadcasted_iota(jnp.int32, (2, NP), 1)
    boxes_ref[0] = jnp.where(lane2 < N_REAL, se, FAR)

    x = cls_ref[0]
    mx = jnp.max(x, axis=0, keepdims=True)
    ex = jnp.exp(x - mx)
    sc = ex / jnp.sum(ex, axis=0, keepdims=True)
    fg = sc[1:NC, :]
    msc = jnp.where(fg > CLS_THRESH, fg, 0.0)
    lane = jax.lax.broadcasted_iota(jnp.int32, (NC - 1, NP), 1)
    msc = jnp.where(lane < N_REAL, msc, 0.0)
    msc_ref[0] = jnp.concatenate(
        [msc, jnp.zeros((C_PAD - (NC - 1), NP), jnp.float32)], axis=0)


def _nms_kernel(bi_ref, mi_ref, bj_ref, mj_ref, out_ref,
                sjb_ref, ejb_ref, ljb_ref, mscb_ref, m_ref, bnd_ref):
    j = pl.program_id(1)
    i = pl.program_id(2)

    @pl.when(i == 0)
    def _():
        sj = bj_ref[0, :, 0:1]
        ej = bj_ref[0, :, 1:2]
        sjb_ref[...] = jnp.broadcast_to(sj, (TJ, TI))
        ejb_ref[...] = jnp.broadcast_to(ej, (TJ, TI))
        ljb_ref[...] = jnp.broadcast_to(
            (ej - sj) * (OVERLAP / (1.0 + OVERLAP)), (TJ, TI))
        ejm = jnp.where(sj < FAR, ej, -FAR)
        bnd_ref[0] = jnp.min(sj)
        bnd_ref[1] = jnp.max(ejm)
        for k in range(NCH):
            bnd_ref[2 + 2 * k] = jnp.min(sj[k * CH:(k + 1) * CH, :])
            bnd_ref[3 + 2 * k] = jnp.max(ejm[k * CH:(k + 1) * CH, :])
        for c in range(NC - 1):
            mscb_ref[c] = jnp.broadcast_to(mj_ref[0, :, c:c + 1], (TJ, TI))

    si = bi_ref[0, 0:1, :]
    ei = bi_ref[0, 1:2, :]

    @pl.when(j == 0)
    def _():
        m_ref[i] = jnp.zeros((C_PAD, TI), jnp.float32)

    ismin = jnp.min(si)
    iemax = jnp.max(jnp.where(si < FAR, ei, -FAR))
    live = (bnd_ref[0] <= iemax) & (bnd_ref[1] >= ismin)

    @pl.when(live)
    def _():
        lis = (ei - si) * (OVERLAP / (1.0 + OVERLAP))

        def chunk_body(k, parts):
            live_k = ((bnd_ref[2 + 2 * k] <= iemax)
                      & (bnd_ref[3 + 2 * k] >= ismin))

            def do(parts):
                r0 = k * CH
                interk = (jnp.minimum(ejb_ref[pl.ds(r0, CH), :], ei)
                          - jnp.maximum(sjb_ref[pl.ds(r0, CH), :], si))
                afk = jnp.where(
                    interk > (ljb_ref[pl.ds(r0, CH), :] + lis), 1.0, 0.0)
                out = []
                for c in range(NC - 1):
                    tk = afk * mscb_ref[c, pl.ds(r0, CH), :]
                    pm = jnp.max(tk.reshape(CH // 8, 8, TI), axis=0)
                    out.append(jnp.maximum(parts[c], pm))
                return tuple(out)

            return jax.lax.cond(live_k, do, lambda p: p, parts)

        parts0 = tuple(jnp.zeros((8, TI), jnp.float32) for _ in range(NC - 1))
        parts = jax.lax.fori_loop(0, NCH, chunk_body, parts0)
        terms = [jnp.max(p, axis=0, keepdims=True) for p in parts]
        new = jnp.concatenate(
            terms + [jnp.zeros((C_PAD - (NC - 1), TI), jnp.float32)], axis=0)
        m_ref[i] = jnp.maximum(m_ref[i], new)

    @pl.when(j == NJ - 1)
    def _():
        m = m_ref[i]
        msci = mi_ref[0]
        contrib = jnp.where(m <= msci, msci, 0.0)
        best = jnp.max(contrib, axis=0, keepdims=True)
        inr = (si > -10.0) & (ei < 10.0)
        best = jnp.where(inr, best, 0.0)
        out_ref[0] = jnp.concatenate(
            [best, jnp.zeros((7, TI), jnp.float32)], axis=0)


def kernel(localizations, classifications, localizations_default):
    B = localizations.shape[0]
    pad = NP - N_REAL

    dc = localizations_default[None, :, 0]
    dw = localizations_default[None, :, 1]
    key = (dc + localizations[:, :, 0] * 0.1 * dw
           - 0.5 * dw * jnp.exp(localizations[:, :, 1] * 0.2))
    key_p = jnp.pad(key, ((0, 0), (0, pad)))
    pos3, perm3 = pl.pallas_call(
        _pos_kernel,
        grid=(B,),
        in_specs=[pl.BlockSpec((1, NP, 1), lambda b: (b, 0, 0))],
        out_specs=[
            pl.BlockSpec((1, NP, 1), lambda b: (b, 0, 0)),
            pl.BlockSpec((1, 1, NP), lambda b: (b, 0, 0)),
        ],
        out_shape=[
            jax.ShapeDtypeStruct((B, NP, 1), jnp.int32),
            jax.ShapeDtypeStruct((B, 1, NP), jnp.int32),
        ],
    )(key_p[:, :, None])
    pos = pos3[:, :, 0]
    perm = perm3[:, 0, :]

    loc_p = jnp.pad(localizations, ((0, 0), (0, pad), (0, 0)))
    cls_p = jnp.pad(classifications, ((0, 0), (0, pad), (0, 0)))
    def_p = jnp.broadcast_to(
        jnp.pad(localizations_default, ((0, pad), (0, 0)))[None], (B, NP, 2))
    loc_s = jnp.take_along_axis(loc_p, perm[:, :, None], axis=1)
    cls_s = jnp.take_along_axis(cls_p, perm[:, :, None], axis=1)
    def_s = jnp.take_along_axis(def_p, perm[:, :, None], axis=1)

    locT = loc_s.transpose(0, 2, 1)
    clsT = cls_s.transpose(0, 2, 1)
    defT = def_s.transpose(0, 2, 1)

    boxes, mscT = pl.pallas_call(
        _prep_kernel,
        grid=(B,),
        in_specs=[
            pl.BlockSpec((1, 2, NP), lambda b: (b, 0, 0)),
            pl.BlockSpec((1, NC, NP), lambda b: (b, 0, 0)),
            pl.BlockSpec((1, 2, NP), lambda b: (b, 0, 0)),
        ],
        out_specs=[
            pl.BlockSpec((1, 2, NP), lambda b: (b, 0, 0)),
            pl.BlockSpec((1, C_PAD, NP), lambda b: (b, 0, 0)),
        ],
        out_shape=[
            jax.ShapeDtypeStruct((B, 2, NP), jnp.float32),
            jax.ShapeDtypeStruct((B, C_PAD, NP), jnp.float32),
        ],
    )(locT, clsT, defT)

    boxes_j = boxes.transpose(0, 2, 1)
    msc_j = mscT.transpose(0, 2, 1)

    best8 = pl.pallas_call(
        _nms_kernel,
        grid=(B, NJ, NI),
        in_specs=[
            pl.BlockSpec((1, 2, TI), lambda b, j, i: (b, 0, i)),
            pl.BlockSpec((1, C_PAD, TI), lambda b, j, i: (b, 0, i)),
            pl.BlockSpec((1, TJ, 2), lambda b, j, i: (b, j, 0)),
            pl.BlockSpec((1, TJ, C_PAD), lambda b, j, i: (b, j, 0)),
        ],
        out_specs=pl.BlockSpec((1, 8, TI), lambda b, j, i: (b, 0, i)),
        out_shape=jax.ShapeDtypeStruct((B, 8, NP), jnp.float32),
        scratch_shapes=[
            pltpu.VMEM((TJ, TI), jnp.float32),
            pltpu.VMEM((TJ, TI), jnp.float32),
            pltpu.VMEM((TJ, TI), jnp.float32),
            pltpu.VMEM((NC - 1, TJ, TI), jnp.float32),
            pltpu.VMEM((NI, C_PAD, TI), jnp.float32),
            pltpu.SMEM((2 + 2 * NCH,), jnp.float32),
        ],
    )(boxes, mscT, boxes_j, msc_j)

    dec = boxes.transpose(0, 2, 1)
    best = best8[:, 0, :]
    out_s = jnp.concatenate([dec, best[:, :, None]], axis=2)
    return jnp.take_along_axis(out_s, pos[:, :N_REAL, None], axis=1)


# speedup vs baseline: 3.7813x; 3.0281x over previous
import jax
import jax.numpy as jnp
from jax.experimental import pallas as pl
from jax.experimental.pallas import tpu as pltpu

N_REAL = 5000
NP = 5120
NC = 11
C_PAD = 16
TI = 128
TJ = 512
CH = 128
NCH = TJ // CH
NI = NP // TI
NJ = NP // TJ
OVERLAP = 0.45
CLS_THRESH = 0.01
FAR = 1e6
NB = 128
THS = OVERLAP / (1.0 + OVERLAP)


def _pos_kernel(key_ref, pos_ref, perm_ref):
    kcol = key_ref[0]
    ridx = jax.lax.broadcasted_iota(jnp.int32, (NP, 1), 0)
    valid = ridx < N_REAL
    kmin = jnp.min(jnp.where(valid, kcol, jnp.inf))
    kmax = jnp.max(jnp.where(valid, kcol, -jnp.inf))
    scale = NB / jnp.maximum(kmax - kmin, 1e-30)
    bfl = jnp.clip(jnp.floor((kcol - kmin) * scale), 0.0, NB - 1.0)
    bi = jnp.where(valid, bfl.astype(jnp.int32), NB - 1)
    lane = jax.lax.broadcasted_iota(jnp.int32, (NP, NB), 1)
    H = jnp.where(lane == bi, 1.0, 0.0)
    C = H
    sh = 1
    while sh < NP:
        C = C + jnp.concatenate(
            [jnp.zeros((sh, NB), jnp.float32), C[:NP - sh, :]], axis=0)
        sh *= 2
    tot = C[NP - 1:NP, :]
    shl = 1
    acc = tot
    while shl < NB:
        acc_shift = jnp.concatenate(
            [jnp.zeros((1, shl), jnp.float32), acc[:, :NB - shl]], axis=1)
        acc = acc + acc_shift
        shl *= 2
    off = acc - tot
    posf = jnp.sum(H * (C + off - 1.0), axis=1, keepdims=True)
    pos_ref[0] = posf.astype(jnp.int32)

    lanef = lane.astype(jnp.float32)
    kf = ridx.astype(jnp.float32)
    gk = jnp.floor(posf * (1.0 / NB))
    lk = posf - gk * NB
    W = jnp.where(lanef == gk, 1.0, 0.0)
    onel = jnp.where(lanef == lk, 1.0, 0.0)
    khi = jnp.floor(kf * (1.0 / 64.0))
    klo = kf - khi * 64.0
    dims = (((0,), (0,)), ((), ()))
    Rhi = jax.lax.dot_general(W, onel * khi, dims,
                              preferred_element_type=jnp.float32)
    Rlo = jax.lax.dot_general(W, onel * klo, dims,
                              preferred_element_type=jnp.float32)
    R = Rhi * 64.0 + Rlo
    permf = R[:NP // NB, :].reshape(1, NP)
    perm_ref[0] = permf.astype(jnp.int32)


def _prep_kernel(loc_ref, cls_ref, def_ref, boxes_ref, msc_ref):
    dc = def_ref[0, 0:1, :]
    dw = def_ref[0, 1:2, :]
    ctr = dc + loc_ref[0, 0:1, :] * 0.1 * dw
    w = dw * jnp.exp(loc_ref[0, 1:2, :] * 0.2)
    s = ctr - w / 2.0
    e = ctr + w / 2.0
    se = jnp.concatenate([s, e], axis=0)
    lane2 = jax.lax.broadcasted_iota(jnp.int32, (2, NP), 1)
    boxes_ref[0] = jnp.where(lane2 < N_REAL, se, FAR)

    x = cls_ref[0]
    mx = jnp.max(x, axis=0, keepdims=True)
    ex = jnp.exp(x - mx)
    sc = ex / jnp.sum(ex, axis=0, keepdims=True)
    fg = sc[1:NC, :]
    msc = jnp.where(fg > CLS_THRESH, fg, 0.0)
    lane = jax.lax.broadcasted_iota(jnp.int32, (NC - 1, NP), 1)
    msc = jnp.where(lane < N_REAL, msc, 0.0)
    msc_ref[0] = jnp.concatenate(
        [msc, jnp.zeros((C_PAD - (NC - 1), NP), jnp.float32)], axis=0)


def _nms_kernel(bi_ref, mi_ref, bj_ref, mj_ref, out_ref,
                sjb_ref, ejb_ref, ljb_ref, mscb_ref, m_ref, ibnd_ref):
    j = pl.program_id(1)

    sj = bj_ref[0, :, 0:1]
    ej = bj_ref[0, :, 1:2]
    sjb_ref[...] = jnp.broadcast_to(sj, (TJ, TI))
    ejb_ref[...] = jnp.broadcast_to(ej, (TJ, TI))
    ljb_ref[...] = jnp.broadcast_to((ej - sj) * THS, (TJ, TI))
    for c in range(NC - 1):
        mscb_ref[c] = jnp.broadcast_to(mj_ref[0, :, c:c + 1], (TJ, TI))
    ejm = jnp.where(sj < FAR, ej, -FAR)
    jb0 = jnp.min(sj)
    jb1 = jnp.max(ejm)

    @pl.when(j == 0)
    def _():
        for i in range(NI):
            sit = bi_ref[0, i, 0:1, :]
            eit = bi_ref[0, i, 1:2, :]
            ibnd_ref[2 * i] = jnp.min(sit)
            ibnd_ref[2 * i + 1] = jnp.max(jnp.where(sit < FAR, eit, -FAR))
        m_ref[...] = jnp.zeros((NI, C_PAD, TI), jnp.float32)

    def i_body(i, carry):
        live = (jb0 <= ibnd_ref[2 * i + 1]) & (jb1 >= ibnd_ref[2 * i])

        @pl.when(live)
        def _():
            si = bi_ref[0, i, 0:1, :]
            ei = bi_ref[0, i, 1:2, :]
            lis = (ei - si) * THS

            def chunk_body(k, parts):
                r0 = k * CH
                interk = (jnp.minimum(ejb_ref[pl.ds(r0, CH), :], ei)
                          - jnp.maximum(sjb_ref[pl.ds(r0, CH), :], si))
                afk = jnp.where(
                    interk > (ljb_ref[pl.ds(r0, CH), :] + lis), 1.0, 0.0)
                out = []
                for c in range(NC - 1):
                    tk = afk * mscb_ref[c, pl.ds(r0, CH), :]
                    pm = jnp.max(tk.reshape(CH // 8, 8, TI), axis=0)
                    out.append(jnp.maximum(parts[c], pm))
                return tuple(out)

            parts0 = tuple(
                jnp.zeros((8, TI), jnp.float32) for _ in range(NC - 1))
            parts = jax.lax.fori_loop(0, NCH, chunk_body, parts0)
            terms = [jnp.max(p, axis=0, keepdims=True) for p in parts]
            new = jnp.concatenate(
                terms + [jnp.zeros((C_PAD - (NC - 1), TI), jnp.float32)],
                axis=0)
            m_ref[i] = jnp.maximum(m_ref[i], new)

        return carry

    jax.lax.fori_loop(0, NI, i_body, 0)

    @pl.when(j == NJ - 1)
    def _():
        def fin_body(i, carry):
            m = m_ref[i]
            msci = mi_ref[0, i]
            contrib = jnp.where(m <= msci, msci, 0.0)
            best = jnp.max(contrib, axis=0, keepdims=True)
            si = bi_ref[0, i, 0:1, :]
            ei = bi_ref[0, i, 1:2, :]
            inr = (si > -10.0) & (ei < 10.0)
            best = jnp.where(inr, best, 0.0)
            out_ref[0, i] = jnp.concatenate(
                [best, jnp.zeros((7, TI), jnp.float32)], axis=0)
            return carry

        jax.lax.fori_loop(0, NI, fin_body, 0)


def kernel(localizations, classifications, localizations_default):
    B = localizations.shape[0]
    pad = NP - N_REAL

    dc = localizations_default[None, :, 0]
    dw = localizations_default[None, :, 1]
    key = (dc + localizations[:, :, 0] * 0.1 * dw
           - 0.5 * dw * jnp.exp(localizations[:, :, 1] * 0.2))
    key_p = jnp.pad(key, ((0, 0), (0, pad)))
    pos3, perm3 = pl.pallas_call(
        _pos_kernel,
        grid=(B,),
        in_specs=[pl.BlockSpec((1, NP, 1), lambda b: (b, 0, 0))],
        out_specs=[
            pl.BlockSpec((1, NP, 1), lambda b: (b, 0, 0)),
            pl.BlockSpec((1, 1, NP), lambda b: (b, 0, 0)),
        ],
        out_shape=[
            jax.ShapeDtypeStruct((B, NP, 1), jnp.int32),
            jax.ShapeDtypeStruct((B, 1, NP), jnp.int32),
        ],
    )(key_p[:, :, None])
    pos = pos3[:, :, 0]
    perm = perm3[:, 0, :]

    loc_p = jnp.pad(localizations, ((0, 0), (0, pad), (0, 0)))
    cls_p = jnp.pad(classifications, ((0, 0), (0, pad), (0, 0)))
    def_p = jnp.broadcast_to(
        jnp.pad(localizations_default, ((0, pad), (0, 0)))[None], (B, NP, 2))
    loc_s = jnp.take_along_axis(loc_p, perm[:, :, None], axis=1)
    cls_s = jnp.take_along_axis(cls_p, perm[:, :, None], axis=1)
    def_s = jnp.take_along_axis(def_p, perm[:, :, None], axis=1)

    locT = loc_s.transpose(0, 2, 1)
    clsT = cls_s.transpose(0, 2, 1)
    defT = def_s.transpose(0, 2, 1)

    boxes, mscT = pl.pallas_call(
        _prep_kernel,
        grid=(B,),
        in_specs=[
            pl.BlockSpec((1, 2, NP), lambda b: (b, 0, 0)),
            pl.BlockSpec((1, NC, NP), lambda b: (b, 0, 0)),
            pl.BlockSpec((1, 2, NP), lambda b: (b, 0, 0)),
        ],
        out_specs=[
            pl.BlockSpec((1, 2, NP), lambda b: (b, 0, 0)),
            pl.BlockSpec((1, C_PAD, NP), lambda b: (b, 0, 0)),
        ],
        out_shape=[
            jax.ShapeDtypeStruct((B, 2, NP), jnp.float32),
            jax.ShapeDtypeStruct((B, C_PAD, NP), jnp.float32),
        ],
    )(locT, clsT, defT)

    boxes_j = boxes.transpose(0, 2, 1)
    msc_j = mscT.transpose(0, 2, 1)
    boxes_i = boxes.reshape(B, 2, NI, TI).transpose(0, 2, 1, 3)
    msc_i = mscT.reshape(B, C_PAD, NI, TI).transpose(0, 2, 1, 3)

    best4 = pl.pallas_call(
        _nms_kernel,
        grid=(B, NJ),
        in_specs=[
            pl.BlockSpec((1, NI, 2, TI), lambda b, j: (b, 0, 0, 0)),
            pl.BlockSpec((1, NI, C_PAD, TI), lambda b, j: (b, 0, 0, 0)),
            pl.BlockSpec((1, TJ, 2), lambda b, j: (b, j, 0)),
            pl.BlockSpec((1, TJ, C_PAD), lambda b, j: (b, j, 0)),
        ],
        out_specs=pl.BlockSpec((1, NI, 8, TI), lambda b, j: (b, 0, 0, 0)),
        out_shape=jax.ShapeDtypeStruct((B, NI, 8, TI), jnp.float32),
        scratch_shapes=[
            pltpu.VMEM((TJ, TI), jnp.float32),
            pltpu.VMEM((TJ, TI), jnp.float32),
            pltpu.VMEM((TJ, TI), jnp.float32),
            pltpu.VMEM((NC - 1, TJ, TI), jnp.float32),
            pltpu.VMEM((NI, C_PAD, TI), jnp.float32),
            pltpu.SMEM((2 * NI,), jnp.float32),
        ],
    )(boxes_i, msc_i, boxes_j, msc_j)

    dec = boxes.transpose(0, 2, 1)
    best = best4[:, :, 0, :].reshape(B, NP)
    out_s = jnp.concatenate([dec, best[:, :, None]], axis=2)
    return jnp.take_along_axis(out_s, pos[:, :N_REAL, None], axis=1)


# speedup vs baseline: 3.9239x; 1.0377x over previous
import jax
import jax.numpy as jnp
from jax.experimental import pallas as pl
from jax.experimental.pallas import tpu as pltpu

N_REAL = 5000
NP = 5120
NC = 11
C_PAD = 16
TI = 128
TJ = 512
CH = 128
NCH = TJ // CH
NI = NP // TI
NJ = NP // TJ
OVERLAP = 0.45
CLS_THRESH = 0.01
FAR = 1e6
NB = 128
THS = OVERLAP / (1.0 + OVERLAP)


def _pos_kernel(key_ref, pos_ref, perm_ref):
    kcol = key_ref[0]
    ridx = jax.lax.broadcasted_iota(jnp.int32, (NP, 1), 0)
    valid = ridx < N_REAL
    kmin = jnp.min(jnp.where(valid, kcol, jnp.inf))
    kmax = jnp.max(jnp.where(valid, kcol, -jnp.inf))
    scale = NB / jnp.maximum(kmax - kmin, 1e-30)
    bfl = jnp.clip(jnp.floor((kcol - kmin) * scale), 0.0, NB - 1.0)
    bi = jnp.where(valid, bfl.astype(jnp.int32), NB - 1)
    lane = jax.lax.broadcasted_iota(jnp.int32, (NP, NB), 1)
    H = jnp.where(lane == bi, 1.0, 0.0)
    C = H
    sh = 1
    while sh < NP:
        C = C + jnp.concatenate(
            [jnp.zeros((sh, NB), jnp.float32), C[:NP - sh, :]], axis=0)
        sh *= 2
    tot = C[NP - 1:NP, :]
    shl = 1
    acc = tot
    while shl < NB:
        acc_shift = jnp.concatenate(
            [jnp.zeros((1, shl), jnp.float32), acc[:, :NB - shl]], axis=1)
        acc = acc + acc_shift
        shl *= 2
    off = acc - tot
    posf = jnp.sum(H * (C + off - 1.0), axis=1, keepdims=True)
    pos_ref[0] = posf.astype(jnp.int32)

    lanef = lane.astype(jnp.float32)
    kf = ridx.astype(jnp.float32)
    gk = jnp.floor(posf * (1.0 / NB))
    lk = posf - gk * NB
    W = jnp.where(lanef == gk, 1.0, 0.0)
    onel = jnp.where(lanef == lk, 1.0, 0.0)
    khi = jnp.floor(kf * (1.0 / 64.0))
    klo = kf - khi * 64.0
    dims = (((0,), (0,)), ((), ()))
    Rhi = jax.lax.dot_general(W, onel * khi, dims,
                              preferred_element_type=jnp.float32)
    Rlo = jax.lax.dot_general(W, onel * klo, dims,
                              preferred_element_type=jnp.float32)
    R = Rhi * 64.0 + Rlo
    permf = R[:NP // NB, :].reshape(1, NP)
    perm_ref[0] = permf.astype(jnp.int32)


def _prep_kernel(loc_ref, cls_ref, def_ref, boxes_ref, msc_ref):
    dc = def_ref[0, 0:1, :]
    dw = def_ref[0, 1:2, :]
    ctr = dc + loc_ref[0, 0:1, :] * 0.1 * dw
    w = dw * jnp.exp(loc_ref[0, 1:2, :] * 0.2)
    s = ctr - w / 2.0
    e = ctr + w / 2.0
    se = jnp.concatenate([s, e], axis=0)
    lane2 = jax.lax.broadcasted_iota(jnp.int32, (2, NP), 1)
    boxes_ref[0] = jnp.where(lane2 < N_REAL, se, FAR)

    x = cls_ref[0]
    mx = jnp.max(x, axis=0, keepdims=True)
    ex = jnp.exp(x - mx)
    sc = ex / jnp.sum(ex, axis=0, keepdims=True)
    fg = sc[1:NC, :]
    msc = jnp.where(fg > CLS_THRESH, fg, 0.0)
    lane = jax.lax.broadcasted_iota(jnp.int32, (NC - 1, NP), 1)
    msc = jnp.where(lane < N_REAL, msc, 0.0)
    msc_ref[0] = jnp.concatenate(
        [msc, jnp.zeros((C_PAD - (NC - 1), NP), jnp.float32)], axis=0)


def _nms_kernel(bi_ref, mi_ref, bj_ref, mj_ref, out_ref,
                sjb_ref, ejb_ref, ljb_ref, mscb_ref, m_ref, ibnd_ref):
    j = pl.program_id(1)

    sj = bj_ref[0, :, 0:1]
    ej = bj_ref[0, :, 1:2]
    sjb_ref[...] = jnp.broadcast_to(sj, (TJ, TI))
    ejb_ref[...] = jnp.broadcast_to(ej, (TJ, TI))
    ljb_ref[...] = jnp.broadcast_to((ej - sj) * THS, (TJ, TI))
    for c in range(NC - 1):
        mscb_ref[c] = jnp.broadcast_to(mj_ref[0, :, c:c + 1], (TJ, TI))
    ejm = jnp.where(sj < FAR, ej, -FAR)
    jb0 = jnp.min(sj)
    jb1 = jnp.max(ejm)
    for k in range(NCH):
        ibnd_ref[2 * NI + 2 * k] = jnp.min(sj[k * CH:(k + 1) * CH, :])
        ibnd_ref[2 * NI + 2 * k + 1] = jnp.max(ejm[k * CH:(k + 1) * CH, :])

    @pl.when(j == 0)
    def _():
        for i in range(NI):
            sit = bi_ref[0, i, 0:1, :]
            eit = bi_ref[0, i, 1:2, :]
            ibnd_ref[2 * i] = jnp.min(sit)
            ibnd_ref[2 * i + 1] = jnp.max(jnp.where(sit < FAR, eit, -FAR))
        m_ref[...] = jnp.zeros((NI, C_PAD, TI), jnp.float32)

    def i_body(i, carry):
        live = (jb0 <= ibnd_ref[2 * i + 1]) & (jb1 >= ibnd_ref[2 * i])

        @pl.when(live)
        def _():
            si = bi_ref[0, i, 0:1, :]
            ei = bi_ref[0, i, 1:2, :]
            lis = (ei - si) * THS
            ism = ibnd_ref[2 * i]
            iem = ibnd_ref[2 * i + 1]

            def chunk_body(k, parts):
                live_k = ((ibnd_ref[2 * NI + 2 * k] <= iem)
                          & (ibnd_ref[2 * NI + 2 * k + 1] >= ism))

                def do(parts):
                    r0 = k * CH
                    interk = (jnp.minimum(ejb_ref[pl.ds(r0, CH), :], ei)
                              - jnp.maximum(sjb_ref[pl.ds(r0, CH), :], si))
                    afk = jnp.where(
                        interk > (ljb_ref[pl.ds(r0, CH), :] + lis), 1.0, 0.0)
                    out = []
                    for c in range(NC - 1):
                        tk = afk * mscb_ref[c, pl.ds(r0, CH), :]
                        pm = jnp.max(tk.reshape(CH // 8, 8, TI), axis=0)
                        out.append(jnp.maximum(parts[c], pm))
                    return tuple(out)

                return jax.lax.cond(live_k, do, lambda p: p, parts)

            parts0 = tuple(
                jnp.zeros((8, TI), jnp.float32) for _ in range(NC - 1))
            parts = jax.lax.fori_loop(0, NCH, chunk_body, parts0)
            terms = [jnp.max(p, axis=0, keepdims=True) for p in parts]
            new = jnp.concatenate(
                terms + [jnp.zeros((C_PAD - (NC - 1), TI), jnp.float32)],
                axis=0)
            m_ref[i] = jnp.maximum(m_ref[i], new)

        return carry

    jax.lax.fori_loop(0, NI, i_body, 0)

    @pl.when(j == NJ - 1)
    def _():
        def fin_body(i, carry):
            m = m_ref[i]
            msci = mi_ref[0, i]
            contrib = jnp.where(m <= msci, msci, 0.0)
            best = jnp.max(contrib, axis=0, keepdims=True)
            si = bi_ref[0, i, 0:1, :]
            ei = bi_ref[0, i, 1:2, :]
            inr = (si > -10.0) & (ei < 10.0)
            best = jnp.where(inr, best, 0.0)
            out_ref[0, i] = jnp.concatenate(
                [best, jnp.zeros((7, TI), jnp.float32)], axis=0)
            return carry

        jax.lax.fori_loop(0, NI, fin_body, 0)


def kernel(localizations, classifications, localizations_default):
    B = localizations.shape[0]
    pad = NP - N_REAL

    dc = localizations_default[None, :, 0]
    dw = localizations_default[None, :, 1]
    key = (dc + localizations[:, :, 0] * 0.1 * dw
           - 0.5 * dw * jnp.exp(localizations[:, :, 1] * 0.2))
    key_p = jnp.pad(key, ((0, 0), (0, pad)))
    pos3, perm3 = pl.pallas_call(
        _pos_kernel,
        grid=(B,),
        in_specs=[pl.BlockSpec((1, NP, 1), lambda b: (b, 0, 0))],
        out_specs=[
            pl.BlockSpec((1, NP, 1), lambda b: (b, 0, 0)),
            pl.BlockSpec((1, 1, NP), lambda b: (b, 0, 0)),
        ],
        out_shape=[
            jax.ShapeDtypeStruct((B, NP, 1), jnp.int32),
            jax.ShapeDtypeStruct((B, 1, NP), jnp.int32),
        ],
    )(key_p[:, :, None])
    pos = pos3[:, :, 0]
    perm = perm3[:, 0, :]

    loc_p = jnp.pad(localizations, ((0, 0), (0, pad), (0, 0)))
    cls_p = jnp.pad(classifications, ((0, 0), (0, pad), (0, 0)))
    def_p = jnp.broadcast_to(
        jnp.pad(localizations_default, ((0, pad), (0, 0)))[None], (B, NP, 2))
    loc_s = jnp.take_along_axis(loc_p, perm[:, :, None], axis=1)
    cls_s = jnp.take_along_axis(cls_p, perm[:, :, None], axis=1)
    def_s = jnp.take_along_axis(def_p, perm[:, :, None], axis=1)

    locT = loc_s.transpose(0, 2, 1)
    clsT = cls_s.transpose(0, 2, 1)
    defT = def_s.transpose(0, 2, 1)

    boxes, mscT = pl.pallas_call(
        _prep_kernel,
        grid=(B,),
        in_specs=[
            pl.BlockSpec((1, 2, NP), lambda b: (b, 0, 0)),
            pl.BlockSpec((1, NC, NP), lambda b: (b, 0, 0)),
            pl.BlockSpec((1, 2, NP), lambda b: (b, 0, 0)),
        ],
        out_specs=[
            pl.BlockSpec((1, 2, NP), lambda b: (b, 0, 0)),
            pl.BlockSpec((1, C_PAD, NP), lambda b: (b, 0, 0)),
        ],
        out_shape=[
            jax.ShapeDtypeStruct((B, 2, NP), jnp.float32),
            jax.ShapeDtypeStruct((B, C_PAD, NP), jnp.float32),
        ],
    )(locT, clsT, defT)

    boxes_j = boxes.transpose(0, 2, 1)
    msc_j = mscT.transpose(0, 2, 1)
    boxes_i = boxes.reshape(B, 2, NI, TI).transpose(0, 2, 1, 3)
    msc_i = mscT.reshape(B, C_PAD, NI, TI).transpose(0, 2, 1, 3)

    best4 = pl.pallas_call(
        _nms_kernel,
        grid=(B, NJ),
        in_specs=[
            pl.BlockSpec((1, NI, 2, TI), lambda b, j: (b, 0, 0, 0)),
            pl.BlockSpec((1, NI, C_PAD, TI), lambda b, j: (b, 0, 0, 0)),
            pl.BlockSpec((1, TJ, 2), lambda b, j: (b, j, 0)),
            pl.BlockSpec((1, TJ, C_PAD), lambda b, j: (b, j, 0)),
        ],
        out_specs=pl.BlockSpec((1, NI, 8, TI), lambda b, j: (b, 0, 0, 0)),
        out_shape=jax.ShapeDtypeStruct((B, NI, 8, TI), jnp.float32),
        scratch_shapes=[
            pltpu.VMEM((TJ, TI), jnp.float32),
            pltpu.VMEM((TJ, TI), jnp.float32),
            pltpu.VMEM((TJ, TI), jnp.float32),
            pltpu.VMEM((NC - 1, TJ, TI), jnp.float32),
            pltpu.VMEM((NI, C_PAD, TI), jnp.float32),
            pltpu.SMEM((2 * NI + 2 * NCH,), jnp.float32),
        ],
    )(boxes_i, msc_i, boxes_j, msc_j)

    dec = boxes.transpose(0, 2, 1)
    best = best4[:, :, 0, :].reshape(B, NP)
    out_s = jnp.concatenate([dec, best[:, :, None]], axis=2)
    return jnp.take_along_axis(out_s, pos[:, :N_REAL, None], axis=1)


# speedup vs baseline: 4.0157x; 1.0234x over previous
import jax
import jax.numpy as jnp
from jax.experimental import pallas as pl
from jax.experimental.pallas import tpu as pltpu

N_REAL = 5000
NP = 5120
NC = 11
C_PAD = 16
TI = 128
TJ = 1024
CH = 128
NCH = TJ // CH
NI = NP // TI
NJ = NP // TJ
OVERLAP = 0.45
CLS_THRESH = 0.01
FAR = 1e6
NB = 128
THS = OVERLAP / (1.0 + OVERLAP)


def _pos_kernel(key_ref, pos_ref, perm_ref):
    kcol = key_ref[0]
    ridx = jax.lax.broadcasted_iota(jnp.int32, (NP, 1), 0)
    valid = ridx < N_REAL
    kmin = jnp.min(jnp.where(valid, kcol, jnp.inf))
    kmax = jnp.max(jnp.where(valid, kcol, -jnp.inf))
    scale = NB / jnp.maximum(kmax - kmin, 1e-30)
    bfl = jnp.clip(jnp.floor((kcol - kmin) * scale), 0.0, NB - 1.0)
    bi = jnp.where(valid, bfl.astype(jnp.int32), NB - 1)
    lane = jax.lax.broadcasted_iota(jnp.int32, (NP, NB), 1)
    H = jnp.where(lane == bi, 1.0, 0.0)
    C = H
    sh = 1
    while sh < NP:
        C = C + jnp.concatenate(
            [jnp.zeros((sh, NB), jnp.float32), C[:NP - sh, :]], axis=0)
        sh *= 2
    tot = C[NP - 1:NP, :]
    shl = 1
    acc = tot
    while shl < NB:
        acc_shift = jnp.concatenate(
            [jnp.zeros((1, shl), jnp.float32), acc[:, :NB - shl]], axis=1)
        acc = acc + acc_shift
        shl *= 2
    off = acc - tot
    posf = jnp.sum(H * (C + off - 1.0), axis=1, keepdims=True)
    pos_ref[0] = posf.astype(jnp.int32)

    lanef = lane.astype(jnp.float32)
    kf = ridx.astype(jnp.float32)
    gk = jnp.floor(posf * (1.0 / NB))
    lk = posf - gk * NB
    W = jnp.where(lanef == gk, 1.0, 0.0)
    onel = jnp.where(lanef == lk, 1.0, 0.0)
    khi = jnp.floor(kf * (1.0 / 64.0))
    klo = kf - khi * 64.0
    dims = (((0,), (0,)), ((), ()))
    Rhi = jax.lax.dot_general(W, onel * khi, dims,
                              preferred_element_type=jnp.float32)
    Rlo = jax.lax.dot_general(W, onel * klo, dims,
                              preferred_element_type=jnp.float32)
    R = Rhi * 64.0 + Rlo
    permf = R[:NP // NB, :].reshape(1, NP)
    perm_ref[0] = permf.astype(jnp.int32)


def _prep_kernel(loc_ref, cls_ref, def_ref, boxes_ref, msc_ref):
    dc = def_ref[0, 0:1, :]
    dw = def_ref[0, 1:2, :]
    ctr = dc + loc_ref[0, 0:1, :] * 0.1 * dw
    w = dw * jnp.exp(loc_ref[0, 1:2, :] * 0.2)
    s = ctr - w / 2.0
    e = ctr + w / 2.0
    se = jnp.concatenate([s, e], axis=0)
    lane2 = jax.lax.broadcasted_iota(jnp.int32, (2, NP), 1)
    boxes_ref[0] = jnp.where(lane2 < N_REAL, se, FAR)

    x = cls_ref[0]
    mx = jnp.max(x, axis=0, keepdims=True)
    ex = jnp.exp(x - mx)
    sc = ex / jnp.sum(ex, axis=0, keepdims=True)
    fg = sc[1:NC, :]
    msc = jnp.where(fg > CLS_THRESH, fg, 0.0)
    lane = jax.lax.broadcasted_iota(jnp.int32, (NC - 1, NP), 1)
    msc = jnp.where(lane < N_REAL, msc, 0.0)
    msc_ref[0] = jnp.concatenate(
        [msc, jnp.zeros((C_PAD - (NC - 1), NP), jnp.float32)], axis=0)


def _nms_kernel(bi_ref, mi_ref, bj_ref, mj_ref, out_ref,
                sjb_ref, ejb_ref, ljb_ref, mscb_ref, m_ref, ibnd_ref):
    j = pl.program_id(1)

    sj = bj_ref[0, :, 0:1]
    ej = bj_ref[0, :, 1:2]
    sjb_ref[...] = jnp.broadcast_to(sj, (TJ, TI))
    ejb_ref[...] = jnp.broadcast_to(ej, (TJ, TI))
    ljb_ref[...] = jnp.broadcast_to((ej - sj) * THS, (TJ, TI))
    for c in range(NC - 1):
        mscb_ref[c] = jnp.broadcast_to(mj_ref[0, :, c:c + 1], (TJ, TI))
    ejm = jnp.where(sj < FAR, ej, -FAR)
    jb0 = jnp.min(sj)
    jb1 = jnp.max(ejm)
    for k in range(NCH):
        ibnd_ref[2 * NI + 2 * k] = jnp.min(sj[k * CH:(k + 1) * CH, :])
        ibnd_ref[2 * NI + 2 * k + 1] = jnp.max(ejm[k * CH:(k + 1) * CH, :])

    @pl.when(j == 0)
    def _():
        for i in range(NI):
            sit = bi_ref[0, i, 0:1, :]
            eit = bi_ref[0, i, 1:2, :]
            ibnd_ref[2 * i] = jnp.min(sit)
            ibnd_ref[2 * i + 1] = jnp.max(jnp.where(sit < FAR, eit, -FAR))
        m_ref[...] = jnp.zeros((NI, C_PAD, TI), jnp.float32)

    def i_body(i, carry):
        live = (jb0 <= ibnd_ref[2 * i + 1]) & (jb1 >= ibnd_ref[2 * i])

        @pl.when(live)
        def _():
            si = bi_ref[0, i, 0:1, :]
            ei = bi_ref[0, i, 1:2, :]
            lis = (ei - si) * THS
            ism = ibnd_ref[2 * i]
            iem = ibnd_ref[2 * i + 1]

            def chunk_body(k, parts):
                live_k = ((ibnd_ref[2 * NI + 2 * k] <= iem)
                          & (ibnd_ref[2 * NI + 2 * k + 1] >= ism))

                def do(parts):
                    r0 = k * CH
                    interk = (jnp.minimum(ejb_ref[pl.ds(r0, CH), :], ei)
                              - jnp.maximum(sjb_ref[pl.ds(r0, CH), :], si))
                    afk = jnp.where(
                        interk > (ljb_ref[pl.ds(r0, CH), :] + lis), 1.0, 0.0)
                    out = []
                    for c in range(NC - 1):
                        tk = afk * mscb_ref[c, pl.ds(r0, CH), :]
                        pm = jnp.max(tk.reshape(CH // 8, 8, TI), axis=0)
                        out.append(jnp.maximum(parts[c], pm))
                    return tuple(out)

                return jax.lax.cond(live_k, do, lambda p: p, parts)

            parts0 = tuple(
                jnp.zeros((8, TI), jnp.float32) for _ in range(NC - 1))
            parts = jax.lax.fori_loop(0, NCH, chunk_body, parts0)
            terms = [jnp.max(p, axis=0, keepdims=True) for p in parts]
            new = jnp.concatenate(
                terms + [jnp.zeros((C_PAD - (NC - 1), TI), jnp.float32)],
                axis=0)
            m_ref[i] = jnp.maximum(m_ref[i], new)

        return carry

    jax.lax.fori_loop(0, NI, i_body, 0)

    @pl.when(j == NJ - 1)
    def _():
        def fin_body(i, carry):
            m = m_ref[i]
            msci = mi_ref[0, i]
            contrib = jnp.where(m <= msci, msci, 0.0)
            best = jnp.max(contrib, axis=0, keepdims=True)
            si = bi_ref[0, i, 0:1, :]
            ei = bi_ref[0, i, 1:2, :]
            inr = (si > -10.0) & (ei < 10.0)
            best = jnp.where(inr, best, 0.0)
            out_ref[0, i] = jnp.concatenate(
                [best, jnp.zeros((7, TI), jnp.float32)], axis=0)
            return carry

        jax.lax.fori_loop(0, NI, fin_body, 0)


def kernel(localizations, classifications, localizations_default):
    B = localizations.shape[0]
    pad = NP - N_REAL

    dc = localizations_default[None, :, 0]
    dw = localizations_default[None, :, 1]
    key = (dc + localizations[:, :, 0] * 0.1 * dw
           - 0.5 * dw * jnp.exp(localizations[:, :, 1] * 0.2))
    key_p = jnp.pad(key, ((0, 0), (0, pad)))
    pos3, perm3 = pl.pallas_call(
        _pos_kernel,
        grid=(B,),
        in_specs=[pl.BlockSpec((1, NP, 1), lambda b: (b, 0, 0))],
        out_specs=[
            pl.BlockSpec((1, NP, 1), lambda b: (b, 0, 0)),
            pl.BlockSpec((1, 1, NP), lambda b: (b, 0, 0)),
        ],
        out_shape=[
            jax.ShapeDtypeStruct((B, NP, 1), jnp.int32),
            jax.ShapeDtypeStruct((B, 1, NP), jnp.int32),
        ],
    )(key_p[:, :, None])
    pos = pos3[:, :, 0]
    perm = perm3[:, 0, :]

    loc_p = jnp.pad(localizations, ((0, 0), (0, pad), (0, 0)))
    cls_p = jnp.pad(classifications, ((0, 0), (0, pad), (0, 0)))
    def_p = jnp.broadcast_to(
        jnp.pad(localizations_default, ((0, pad), (0, 0)))[None], (B, NP, 2))
    loc_s = jnp.take_along_axis(loc_p, perm[:, :, None], axis=1)
    cls_s = jnp.take_along_axis(cls_p, perm[:, :, None], axis=1)
    def_s = jnp.take_along_axis(def_p, perm[:, :, None], axis=1)

    locT = loc_s.transpose(0, 2, 1)
    clsT = cls_s.transpose(0, 2, 1)
    defT = def_s.transpose(0, 2, 1)

    boxes, mscT = pl.pallas_call(
        _prep_kernel,
        grid=(B,),
        in_specs=[
            pl.BlockSpec((1, 2, NP), lambda b: (b, 0, 0)),
            pl.BlockSpec((1, NC, NP), lambda b: (b, 0, 0)),
            pl.BlockSpec((1, 2, NP), lambda b: (b, 0, 0)),
        ],
        out_specs=[
            pl.BlockSpec((1, 2, NP), lambda b: (b, 0, 0)),
            pl.BlockSpec((1, C_PAD, NP), lambda b: (b, 0, 0)),
        ],
        out_shape=[
            jax.ShapeDtypeStruct((B, 2, NP), jnp.float32),
            jax.ShapeDtypeStruct((B, C_PAD, NP), jnp.float32),
        ],
    )(locT, clsT, defT)

    boxes_j = boxes.transpose(0, 2, 1)
    msc_j = mscT.transpose(0, 2, 1)
    boxes_i = boxes.reshape(B, 2, NI, TI).transpose(0, 2, 1, 3)
    msc_i = mscT.reshape(B, C_PAD, NI, TI).transpose(0, 2, 1, 3)

    best4 = pl.pallas_call(
        _nms_kernel,
        grid=(B, NJ),
        in_specs=[
            pl.BlockSpec((1, NI, 2, TI), lambda b, j: (b, 0, 0, 0)),
            pl.BlockSpec((1, NI, C_PAD, TI), lambda b, j: (b, 0, 0, 0)),
            pl.BlockSpec((1, TJ, 2), lambda b, j: (b, j, 0)),
            pl.BlockSpec((1, TJ, C_PAD), lambda b, j: (b, j, 0)),
        ],
        out_specs=pl.BlockSpec((1, NI, 8, TI), lambda b, j: (b, 0, 0, 0)),
        out_shape=jax.ShapeDtypeStruct((B, NI, 8, TI), jnp.float32),
        scratch_shapes=[
            pltpu.VMEM((TJ, TI), jnp.float32),
            pltpu.VMEM((TJ, TI), jnp.float32),
            pltpu.VMEM((TJ, TI), jnp.float32),
            pltpu.VMEM((NC - 1, TJ, TI), jnp.float32),
            pltpu.VMEM((NI, C_PAD, TI), jnp.float32),
            pltpu.SMEM((2 * NI + 2 * NCH,), jnp.float32),
        ],
    )(boxes_i, msc_i, boxes_j, msc_j)

    dec = boxes.transpose(0, 2, 1)
    best = best4[:, :, 0, :].reshape(B, NP)
    out_s = jnp.concatenate([dec, best[:, :, None]], axis=2)
    return jnp.take_along_axis(out_s, pos[:, :N_REAL, None], axis=1)
